# Initial kernel scaffold; baseline (speedup 1.0000x reference)
import jax
import jax.numpy as jnp
from jax.experimental import pallas as pl


def kernel(x, table):
    raise NotImplementedError("write your pallas kernel here")


# speedup vs baseline: 1.8175x; 1.8175x over previous
import functools

import jax
import jax.numpy as jnp
from jax import lax
from jax.experimental import pallas as pl
from jax.experimental.pallas import tpu as pltpu
from jax.experimental.pallas import tpu_sc as plsc

VOCAB = 1_000_000
D = 64
BATCH = 16384
HIST = 50
B = BATCH * HIST

NC = 2
NS = 16
NW = NC * NS
BPW = B // NW

IPS = 128
K = 5
CH = K * IPS
NCHUNK = BPW // CH

_mesh = plsc.VectorSubcoreMesh(core_axis_name="c", subcore_axis_name="s")


@functools.partial(
    pl.kernel,
    mesh=_mesh,
    out_type=jax.ShapeDtypeStruct((B, D), jnp.float32),
    scratch_types=[
        pltpu.VMEM((CH,), jnp.int32),
        pltpu.VMEM((CH, D), jnp.float32),
        pltpu.SemaphoreType.DMA,
    ],
    compiler_params=pltpu.CompilerParams(use_tc_tiling_on_sc=False),
)
def _emb_lookup(idx_hbm, table_hbm, out_hbm, idx_v, rows_v, sem):
    wid = lax.axis_index("s") * NC + lax.axis_index("c")
    base = wid * BPW

    def body(c, carry):
        off = base + c * CH
        pltpu.sync_copy(idx_hbm.at[pl.ds(off, CH)], idx_v)
        cps = [
            pltpu.async_copy(
                table_hbm.at[idx_v.at[pl.ds(j * IPS, IPS)]],
                rows_v.at[pl.ds(j * IPS, IPS)],
                sem,
            )
            for j in range(K)
        ]
        for cp in cps:
            cp.wait()
        pltpu.sync_copy(rows_v, out_hbm.at[pl.ds(off, CH)])
        return carry

    lax.fori_loop(0, NCHUNK, body, 0)


def kernel(x, table):
    idx = x.reshape(B)
    out = _emb_lookup(idx, table)
    return out.reshape(BATCH, HIST, D)


# speedup vs baseline: 1.8615x; 1.0242x over previous
import functools

import jax
import jax.numpy as jnp
from jax import lax
from jax.experimental import pallas as pl
from jax.experimental.pallas import tpu as pltpu
from jax.experimental.pallas import tpu_sc as plsc

VOCAB = 1_000_000
D = 64
BATCH = 16384
HIST = 50
B = BATCH * HIST

NC = 2
NS = 16
NW = NC * NS
BPW = B // NW

IPS = 128
K = 5
CH = K * IPS
NCHUNK = BPW // CH

NBUF = 2

_mesh = plsc.VectorSubcoreMesh(core_axis_name="c", subcore_axis_name="s")


@functools.partial(
    pl.kernel,
    mesh=_mesh,
    out_type=jax.ShapeDtypeStruct((B, D), jnp.float32),
    scratch_types=[
        pltpu.VMEM((NBUF * CH,), jnp.int32),
        pltpu.VMEM((NBUF * CH, D), jnp.float32),
        pltpu.SemaphoreType.DMA,
        pltpu.SemaphoreType.DMA,
    ],
    compiler_params=pltpu.CompilerParams(use_tc_tiling_on_sc=False),
)
def _emb_lookup(idx_hbm, table_hbm, out_hbm, idx_v, rows_v, gsem, osem):
    wid = lax.axis_index("s") * NC + lax.axis_index("c")
    base = wid * BPW

    def fire(g, b):
        off = base + g * CH
        pltpu.sync_copy(idx_hbm.at[pl.ds(off, CH)], idx_v.at[pl.ds(b * CH, CH)])
        for j in range(K):
            pltpu.async_copy(
                table_hbm.at[idx_v.at[pl.ds(b * CH + j * IPS, IPS)]],
                rows_v.at[pl.ds(b * CH + j * IPS, IPS)],
                gsem,
            )

    def wait_gathers(b):
        pltpu.make_async_copy(
            out_hbm.at[pl.ds(base, CH)], rows_v.at[pl.ds(b * CH, CH)], gsem
        ).wait()

    def store(g, b):
        off = base + g * CH
        pltpu.async_copy(
            rows_v.at[pl.ds(b * CH, CH)], out_hbm.at[pl.ds(off, CH)], osem
        )

    def wait_store():
        pltpu.make_async_copy(
            rows_v.at[pl.ds(0, CH)], out_hbm.at[pl.ds(base, CH)], osem
        ).wait()

    fire(0, 0)

    def body(g, carry):
        b = lax.rem(g, NBUF)
        pb = 1 - b

        @pl.when(g >= 2)
        def _():
            wait_store()

        fire(g, b)
        wait_gathers(pb)
        store(g - 1, pb)
        return carry

    lax.fori_loop(1, NCHUNK, body, 0)

    last = NCHUNK - 1
    wait_gathers(last % NBUF)
    store(last, last % NBUF)
    wait_store()
    wait_store()


def kernel(x, table):
    idx = x.reshape(B)
    out = _emb_lookup(idx, table)
    return out.reshape(BATCH, HIST, D)
